# baseline (speedup 1.0000x reference)
import jax
import jax.numpy as jnp
from jax import lax
from jax.experimental import pallas as pl
from jax.experimental.pallas import tpu as pltpu
from jax.experimental.pallas import tpu_sc as plsc

_D = 32
_C = 20
_NC, _NS = 2, 16
_NW = _NC * _NS
_SZ = 800
_L = 16


def _sc_body(x_hbm, table_hbm, out_hbm, xraw_v, idx_v, acc_v,
             rsem, isem, gsem, g0sem, osem):
    wid = lax.axis_index("s") * _NC + lax.axis_index("c")
    n_rows = out_hbm.shape[0]
    per_w = n_rows // _NW
    chunks = per_w // _SZ
    lane = lax.iota(jnp.int32, _L)

    def fire_raw(i):
        return pltpu.async_copy(
            x_hbm.at[pl.ds(wid * per_w + i * _SZ, _SZ), :], xraw_v.at[i % 2],
            rsem.at[i % 2],
        )

    def fire_out(i):
        return pltpu.async_copy(
            acc_v.at[i % 2], out_hbm.at[pl.ds(wid * per_w + i * _SZ, _SZ)],
            osem.at[i % 2],
        )

    def transpose_chunk(b):
        raw = xraw_v.at[b]
        dst = idx_v.at[b]

        def step(r16, carry):
            rows = lane + r16 * _L
            for c in range(_C):
                vals = plsc.load_gather(raw, [rows, jnp.full((_L,), c, jnp.int32)])
                dst[c, pl.ds(r16 * _L, _L)] = vals
            return carry

        lax.fori_loop(0, _SZ // _L, step, 0)

    out_cp = [None] * chunks
    raw_cp = [None] * chunks
    adds_prev = None
    raw_cp[0] = fire_raw(0)
    for i in range(chunks):
        b = i % 2
        if i >= 2:
            out_cp[i - 2].wait()
        raw_cp[i].wait()
        transpose_chunk(b)
        g0 = pltpu.async_copy(table_hbm.at[idx_v.at[b].at[0]], acc_v.at[b],
                              g0sem.at[b])
        if adds_prev is not None:
            for cp in adds_prev:
                cp.wait()
            out_cp[i - 1] = fire_out(i - 1)
        if i + 1 < chunks:
            raw_cp[i + 1] = fire_raw(i + 1)
        g0.wait()
        adds_prev = [
            pltpu.async_copy(table_hbm.at[idx_v.at[b].at[c]], acc_v.at[b],
                             gsem.at[b], add=True)
            for c in range(1, _C)
        ]
    for cp in adds_prev:
        cp.wait()
    out_cp[chunks - 1] = fire_out(chunks - 1)
    out_cp[chunks - 2].wait()
    out_cp[chunks - 1].wait()


def kernel(x, table):
    b, v, c = x.shape
    n = b * v
    run = pl.kernel(
        _sc_body,
        out_type=jax.ShapeDtypeStruct((n, _D), jnp.float32),
        mesh=plsc.VectorSubcoreMesh(core_axis_name="c", subcore_axis_name="s"),
        scratch_types=[
            pltpu.VMEM((2, _SZ, _C), jnp.int32),
            pltpu.VMEM((2, _C, _SZ), jnp.int32),
            pltpu.VMEM((2, _SZ, _D), jnp.float32),
            pltpu.SemaphoreType.DMA((2,)),
            pltpu.SemaphoreType.DMA((2,)),
            pltpu.SemaphoreType.DMA((2,)),
            pltpu.SemaphoreType.DMA((2,)),
            pltpu.SemaphoreType.DMA((2,)),
        ],
        compiler_params=pltpu.CompilerParams(
            use_tc_tiling_on_sc=False, needs_layout_passes=False
        ),
    )
    out = run(x.reshape(n, c), table)
    return out.reshape(b, v, _D)


# speedup vs baseline: 1.0025x; 1.0025x over previous
import jax
import jax.numpy as jnp
from jax import lax
from jax.experimental import pallas as pl
from jax.experimental.pallas import tpu as pltpu
from jax.experimental.pallas import tpu_sc as plsc

_D = 32
_C = 20
_NC, _NS = 2, 16
_NW = _NC * _NS
_SZ = 800
_L = 16


def _sc_body(x_hbm, table_hbm, out_hbm, xraw_v, idx_v, acc_v,
             rsem, gsem, g0sem, osem):
    wid = lax.axis_index("s") * _NC + lax.axis_index("c")
    n_rows = out_hbm.shape[0]
    per_w = n_rows // _NW
    chunks = per_w // _SZ
    lane = lax.iota(jnp.int32, _L)

    def fire_raw(i):
        return pltpu.async_copy(
            x_hbm.at[pl.ds(wid * per_w + i * _SZ, _SZ), :], xraw_v.at[i % 2],
            rsem.at[i % 2],
        )

    def fire_out(i):
        return pltpu.async_copy(
            acc_v.at[i % 2], out_hbm.at[pl.ds(wid * per_w + i * _SZ, _SZ)],
            osem.at[i % 2],
        )

    def transpose_chunk(b):
        raw = xraw_v.at[b]
        dst = idx_v.at[b]

        @plsc.parallel_loop(0, _SZ // _L, unroll=4)
        def step(r16):
            rows = lane + r16 * _L
            for c in range(_C):
                vals = plsc.load_gather(raw, [rows, jnp.full((_L,), c, jnp.int32)])
                dst[c, pl.ds(r16 * _L, _L)] = vals

    out_cp = [None] * chunks
    raw_cp = [None] * chunks
    adds_prev = None
    raw_cp[0] = fire_raw(0)
    for i in range(chunks):
        b = i % 2
        if i >= 2:
            out_cp[i - 2].wait()
        raw_cp[i].wait()
        transpose_chunk(b)
        g0 = pltpu.async_copy(table_hbm.at[idx_v.at[b].at[0]], acc_v.at[b],
                              g0sem.at[b])
        if adds_prev is not None:
            for cp in adds_prev:
                cp.wait()
            out_cp[i - 1] = fire_out(i - 1)
        if i + 1 < chunks:
            raw_cp[i + 1] = fire_raw(i + 1)
        g0.wait()
        adds_prev = [
            pltpu.async_copy(table_hbm.at[idx_v.at[b].at[c]], acc_v.at[b],
                             gsem.at[b], add=True)
            for c in range(1, _C)
        ]
    for cp in adds_prev:
        cp.wait()
    out_cp[chunks - 1] = fire_out(chunks - 1)
    out_cp[chunks - 2].wait()
    out_cp[chunks - 1].wait()


def kernel(x, table):
    b, v, c = x.shape
    n = b * v
    run = pl.kernel(
        _sc_body,
        out_type=jax.ShapeDtypeStruct((n, _D), jnp.float32),
        mesh=plsc.VectorSubcoreMesh(core_axis_name="c", subcore_axis_name="s"),
        scratch_types=[
            pltpu.VMEM((2, _SZ, _C), jnp.int32),
            pltpu.VMEM((2, _C, _SZ), jnp.int32),
            pltpu.VMEM((2, _SZ, _D), jnp.float32),
            pltpu.SemaphoreType.DMA((2,)),
            pltpu.SemaphoreType.DMA((2,)),
            pltpu.SemaphoreType.DMA((2,)),
            pltpu.SemaphoreType.DMA((2,)),
        ],
        compiler_params=pltpu.CompilerParams(
            use_tc_tiling_on_sc=False, needs_layout_passes=False
        ),
    )
    out = run(x.reshape(n, c), table)
    return out.reshape(b, v, _D)
